# Initial kernel scaffold; baseline (speedup 1.0000x reference)
import jax
import jax.numpy as jnp
from jax.experimental import pallas as pl


def kernel(x, edge_index, W1, b1, W2, b2):
    raise NotImplementedError("write your pallas kernel here")


# speedup vs baseline: 18.8369x; 18.8369x over previous
import functools

import jax
import jax.numpy as jnp
from jax import lax
from jax.experimental import pallas as pl
from jax.experimental.pallas import tpu as pltpu
from jax.experimental.pallas import tpu_sc as plsc

N = 10000
E = 320000
D_IN, D_HID, D_OUT = 128, 32, 16

NC, NS = 2, 16
NW = NC * NS
CHUNK = 128
NCHUNK = 79
EPW = CHUNK * NCHUNK
E_PAD = EPW * NW
ROWS = 632
ACC_N = ROWS * NS
TAIL = N - (NS - 1) * ROWS
DEGW = 8

_mesh = plsc.VectorSubcoreMesh(core_axis_name="c", subcore_axis_name="s")
_sc_params = pltpu.CompilerParams(use_tc_tiling_on_sc=False)


def _make_deg_kernel():
  @functools.partial(
      pl.kernel,
      out_type=jax.ShapeDtypeStruct((NC, N, DEGW), jnp.float32),
      mesh=_mesh,
      compiler_params=_sc_params,
      scratch_types=[
          pltpu.VMEM((CHUNK,), jnp.int32),
          pltpu.VMEM((CHUNK, DEGW), jnp.float32),
          pltpu.VMEM_SHARED((ACC_N, DEGW), jnp.float32),
      ],
  )
  def deg_kernel(dst_hbm, ones_hbm, zeros_hbm, out_hbm, dst_v, ones_v, acc):
    c = lax.axis_index("c")
    s = lax.axis_index("s")
    wid = s * NC + c
    pltpu.sync_copy(zeros_hbm, acc.at[pl.ds(s * ROWS, ROWS)])
    pltpu.sync_copy(ones_hbm, ones_v)
    plsc.subcore_barrier()

    def body(j, carry):
      off = wid * EPW + j * CHUNK
      pltpu.sync_copy(dst_hbm.at[pl.ds(off, CHUNK)], dst_v)
      pltpu.sync_copy(ones_v, acc.at[dst_v], add=True)
      return carry

    lax.fori_loop(0, NCHUNK, body, 0)
    plsc.subcore_barrier()

    @pl.when(s < NS - 1)
    def _():
      pltpu.sync_copy(acc.at[pl.ds(s * ROWS, ROWS)],
                      out_hbm.at[c, pl.ds(s * ROWS, ROWS)])

    @pl.when(s == NS - 1)
    def _():
      pltpu.sync_copy(acc.at[pl.ds((NS - 1) * ROWS, TAIL)],
                      out_hbm.at[c, pl.ds((NS - 1) * ROWS, TAIL)])

  return deg_kernel


def _make_scatter_kernel(d: int):
  @functools.partial(
      pl.kernel,
      out_type=jax.ShapeDtypeStruct((NC, N, d), jnp.float32),
      mesh=_mesh,
      compiler_params=_sc_params,
      scratch_types=[
          pltpu.VMEM((CHUNK,), jnp.int32),
          pltpu.VMEM((CHUNK,), jnp.int32),
          pltpu.VMEM((CHUNK, d), jnp.float32),
          pltpu.VMEM_SHARED((ACC_N, d), jnp.float32),
      ],
  )
  def scat_kernel(z_hbm, src_hbm, dst_hbm, zeros_hbm, out_hbm,
                  src_v, dst_v, rows_v, acc):
    c = lax.axis_index("c")
    s = lax.axis_index("s")
    wid = s * NC + c
    pltpu.sync_copy(zeros_hbm, acc.at[pl.ds(s * ROWS, ROWS)])
    plsc.subcore_barrier()

    def body(j, carry):
      off = wid * EPW + j * CHUNK
      pltpu.sync_copy(src_hbm.at[pl.ds(off, CHUNK)], src_v)
      pltpu.sync_copy(dst_hbm.at[pl.ds(off, CHUNK)], dst_v)
      pltpu.sync_copy(z_hbm.at[src_v], rows_v)
      pltpu.sync_copy(rows_v, acc.at[dst_v], add=True)
      return carry

    lax.fori_loop(0, NCHUNK, body, 0)
    plsc.subcore_barrier()

    @pl.when(s < NS - 1)
    def _():
      pltpu.sync_copy(acc.at[pl.ds(s * ROWS, ROWS)],
                      out_hbm.at[c, pl.ds(s * ROWS, ROWS)])

    @pl.when(s == NS - 1)
    def _():
      pltpu.sync_copy(acc.at[pl.ds((NS - 1) * ROWS, TAIL)],
                      out_hbm.at[c, pl.ds((NS - 1) * ROWS, TAIL)])

  return scat_kernel


_deg_kernel = _make_deg_kernel()
_scat32 = _make_scatter_kernel(D_HID)
_scat16 = _make_scatter_kernel(D_OUT)

_HIGH = lax.Precision.HIGHEST


def _tc1(x_ref, w1_ref, deg_ref, z1_ref, dinv_ref):
  deg = deg_ref[0, :, :1] + deg_ref[1, :, :1] + 1.0
  dinv = lax.rsqrt(deg)
  y1 = jnp.dot(x_ref[...], w1_ref[...],
               preferred_element_type=jnp.float32, precision=_HIGH)
  z1_ref[...] = y1 * dinv
  dinv_ref[...] = dinv


def _tc2(s1_ref, z1_ref, dinv_ref, b1_ref, w2_ref, z2_ref):
  dinv = dinv_ref[...]
  h = dinv * (s1_ref[0] + s1_ref[1] + z1_ref[...]) + b1_ref[...]
  h = jnp.maximum(h, 0.0)
  y2 = jnp.dot(h, w2_ref[...],
               preferred_element_type=jnp.float32, precision=_HIGH)
  z2_ref[...] = y2 * dinv


def _tc3(s2_ref, z2_ref, dinv_ref, b2_ref, out_ref):
  out_ref[...] = (dinv_ref[...] * (s2_ref[0] + s2_ref[1] + z2_ref[...])
                  + b2_ref[...])


def kernel(x, edge_index, W1, b1, W2, b2):
  src = edge_index[0]
  dst = edge_index[1]
  pad = E_PAD - E
  src_p = jnp.concatenate([src, jnp.zeros((pad,), jnp.int32)])
  dst_p = jnp.concatenate([dst, jnp.full((pad,), N, jnp.int32)])

  ones_c = jnp.ones((CHUNK, DEGW), jnp.float32)
  zeros1 = jnp.zeros((ROWS, DEGW), jnp.float32)
  zeros32 = jnp.zeros((ROWS, D_HID), jnp.float32)
  zeros16 = jnp.zeros((ROWS, D_OUT), jnp.float32)

  deg_parts = _deg_kernel(dst_p, ones_c, zeros1)

  z1, dinv = pl.pallas_call(
      _tc1,
      out_shape=(jax.ShapeDtypeStruct((N, D_HID), jnp.float32),
                 jax.ShapeDtypeStruct((N, 1), jnp.float32)),
  )(x, W1, deg_parts)

  s1_parts = _scat32(z1, src_p, dst_p, zeros32)

  z2 = pl.pallas_call(
      _tc2,
      out_shape=jax.ShapeDtypeStruct((N, D_OUT), jnp.float32),
  )(s1_parts, z1, dinv, b1, W2)

  s2_parts = _scat16(z2, src_p, dst_p, zeros16)

  out = pl.pallas_call(
      _tc3,
      out_shape=jax.ShapeDtypeStruct((N, D_OUT), jnp.float32),
  )(s2_parts, z2, dinv, b2)

  return out


# speedup vs baseline: 30.8708x; 1.6388x over previous
import functools

import jax
import jax.numpy as jnp
from jax import lax
from jax.experimental import pallas as pl
from jax.experimental.pallas import tpu as pltpu
from jax.experimental.pallas import tpu_sc as plsc

N = 10000
E = 320000
D_IN, D_HID, D_OUT = 128, 32, 16

NC, NS = 2, 16
NW = NC * NS
CHUNK = 128
NBUF = 8
NG = 10
NCHUNK = NBUF * NG
EPW = CHUNK * NCHUNK
E_PAD = EPW * NW
ROWS = 632
ACC_N = ROWS * NS
TAIL = N - (NS - 1) * ROWS
DEGW = 8

_mesh = plsc.VectorSubcoreMesh(core_axis_name="c", subcore_axis_name="s")
_sc_params = pltpu.CompilerParams(use_tc_tiling_on_sc=False)


def _copy_out(acc, out_hbm, c, s):
  @pl.when(s < NS - 1)
  def _():
    pltpu.sync_copy(acc.at[pl.ds(s * ROWS, ROWS)],
                    out_hbm.at[c, pl.ds(s * ROWS, ROWS)])

  @pl.when(s == NS - 1)
  def _():
    pltpu.sync_copy(acc.at[pl.ds((NS - 1) * ROWS, TAIL)],
                    out_hbm.at[c, pl.ds((NS - 1) * ROWS, TAIL)])


def _make_deg_kernel():
  @functools.partial(
      pl.kernel,
      out_type=jax.ShapeDtypeStruct((NC, N, DEGW), jnp.float32),
      mesh=_mesh,
      compiler_params=_sc_params,
      name="sc_deg",
      scratch_types=[
          pltpu.VMEM((NCHUNK, CHUNK), jnp.int32),
          pltpu.VMEM((CHUNK, DEGW), jnp.float32),
          pltpu.VMEM_SHARED((ACC_N, DEGW), jnp.float32),
          pltpu.SemaphoreType.DMA,
      ],
  )
  def deg_kernel(dst_hbm, ones_hbm, zeros_hbm, out_hbm, dst_v, ones_v, acc,
                 ssem):
    c = lax.axis_index("c")
    s = lax.axis_index("s")
    wid = s * NC + c
    pltpu.sync_copy(zeros_hbm, acc.at[pl.ds(s * ROWS, ROWS)])
    pltpu.sync_copy(ones_hbm, ones_v)
    pltpu.sync_copy(dst_hbm.at[pl.ds(wid * NCHUNK, NCHUNK)], dst_v)
    plsc.subcore_barrier()

    def body(g, carry):
      for b in range(NBUF):
        pltpu.async_copy(ones_v, acc.at[dst_v.at[g * NBUF + b]], ssem,
                         add=True)
      for b in range(NBUF):
        pltpu.make_async_copy(ones_v, acc.at[pl.ds(0, CHUNK)], ssem).wait()
      return carry

    lax.fori_loop(0, NG, body, 0)
    plsc.subcore_barrier()
    _copy_out(acc, out_hbm, c, s)

  return deg_kernel


def _make_scatter_kernel(d: int):
  @functools.partial(
      pl.kernel,
      out_type=jax.ShapeDtypeStruct((NC, N, d), jnp.float32),
      mesh=_mesh,
      compiler_params=_sc_params,
      name=f"sc_scat{d}",
      scratch_types=[
          pltpu.VMEM((NCHUNK, CHUNK), jnp.int32),
          pltpu.VMEM((NCHUNK, CHUNK), jnp.int32),
          pltpu.VMEM((NBUF, CHUNK, d), jnp.float32),
          pltpu.VMEM_SHARED((ACC_N, d), jnp.float32),
      ] + [pltpu.SemaphoreType.DMA] * (2 * NBUF),
  )
  def scat_kernel(z_hbm, src_hbm, dst_hbm, zeros_hbm, out_hbm,
                  src_v, dst_v, rows_v, acc, *sems):
    gsems, ssems = sems[:NBUF], sems[NBUF:]
    c = lax.axis_index("c")
    s = lax.axis_index("s")
    wid = s * NC + c
    pltpu.sync_copy(zeros_hbm, acc.at[pl.ds(s * ROWS, ROWS)])
    pltpu.sync_copy(src_hbm.at[pl.ds(wid * NCHUNK, NCHUNK)], src_v)
    pltpu.sync_copy(dst_hbm.at[pl.ds(wid * NCHUNK, NCHUNK)], dst_v)
    plsc.subcore_barrier()

    def gather_start(b, j):
      pltpu.async_copy(z_hbm.at[src_v.at[j]], rows_v.at[b], gsems[b])

    def gather_wait(b):
      pltpu.make_async_copy(z_hbm.at[pl.ds(0, CHUNK)], rows_v.at[b],
                            gsems[b]).wait()

    def scatter_start(b, j):
      pltpu.async_copy(rows_v.at[b], acc.at[dst_v.at[j]], ssems[b], add=True)

    def scatter_wait(b):
      pltpu.make_async_copy(rows_v.at[b], acc.at[pl.ds(0, CHUNK)],
                            ssems[b]).wait()

    for b in range(NBUF):
      gather_start(b, b)

    def body(g, carry):
      for b in range(NBUF):
        gather_wait(b)
        scatter_start(b, g * NBUF + b)
      for b in range(NBUF):
        scatter_wait(b)
        gather_start(b, (g + 1) * NBUF + b)
      return carry

    lax.fori_loop(0, NG - 1, body, 0)
    for b in range(NBUF):
      gather_wait(b)
      scatter_start(b, (NG - 1) * NBUF + b)
    for b in range(NBUF):
      scatter_wait(b)
    plsc.subcore_barrier()
    _copy_out(acc, out_hbm, c, s)

  return scat_kernel


_deg_kernel = _make_deg_kernel()
_scat32 = _make_scatter_kernel(D_HID)
_scat16 = _make_scatter_kernel(D_OUT)

_HIGH = lax.Precision.HIGHEST


def _tc1(x_ref, w1_ref, deg_ref, z1_ref, dinv_ref):
  deg = deg_ref[0, :, :1] + deg_ref[1, :, :1] + 1.0
  dinv = lax.rsqrt(deg)
  y1 = jnp.dot(x_ref[...], w1_ref[...],
               preferred_element_type=jnp.float32, precision=_HIGH)
  z1_ref[...] = y1 * dinv
  dinv_ref[...] = dinv


def _tc2(s1_ref, z1_ref, dinv_ref, b1_ref, w2_ref, z2_ref):
  dinv = dinv_ref[...]
  h = dinv * (s1_ref[0] + s1_ref[1] + z1_ref[...]) + b1_ref[...]
  h = jnp.maximum(h, 0.0)
  y2 = jnp.dot(h, w2_ref[...],
               preferred_element_type=jnp.float32, precision=_HIGH)
  z2_ref[...] = y2 * dinv


def _tc3(s2_ref, z2_ref, dinv_ref, b2_ref, out_ref):
  out_ref[...] = (dinv_ref[...] * (s2_ref[0] + s2_ref[1] + z2_ref[...])
                  + b2_ref[...])


def kernel(x, edge_index, W1, b1, W2, b2):
  src = edge_index[0]
  dst = edge_index[1]
  pad = E_PAD - E
  src_p = jnp.concatenate([src, jnp.zeros((pad,), jnp.int32)])
  dst_p = jnp.concatenate([dst, jnp.full((pad,), N, jnp.int32)])
  src_b = src_p.reshape(NW * NCHUNK, CHUNK)
  dst_b = dst_p.reshape(NW * NCHUNK, CHUNK)

  ones_c = jnp.ones((CHUNK, DEGW), jnp.float32)
  zeros1 = jnp.zeros((ROWS, DEGW), jnp.float32)
  zeros32 = jnp.zeros((ROWS, D_HID), jnp.float32)
  zeros16 = jnp.zeros((ROWS, D_OUT), jnp.float32)

  deg_parts = _deg_kernel(dst_b, ones_c, zeros1)

  z1, dinv = pl.pallas_call(
      _tc1,
      out_shape=(jax.ShapeDtypeStruct((N, D_HID), jnp.float32),
                 jax.ShapeDtypeStruct((N, 1), jnp.float32)),
  )(x, W1, deg_parts)

  s1_parts = _scat32(z1, src_b, dst_b, zeros32)

  z2 = pl.pallas_call(
      _tc2,
      out_shape=jax.ShapeDtypeStruct((N, D_OUT), jnp.float32),
  )(s1_parts, z1, dinv, b1, W2)

  s2_parts = _scat16(z2, src_b, dst_b, zeros16)

  out = pl.pallas_call(
      _tc3,
      out_shape=jax.ShapeDtypeStruct((N, D_OUT), jnp.float32),
  )(s2_parts, z2, dinv, b2)

  return out


# speedup vs baseline: 32.3325x; 1.0474x over previous
import functools

import jax
import jax.numpy as jnp
from jax import lax
from jax.experimental import pallas as pl
from jax.experimental.pallas import tpu as pltpu
from jax.experimental.pallas import tpu_sc as plsc

N = 10000
E = 320000
D_IN, D_HID, D_OUT = 128, 32, 16

NC, NS = 2, 16
NW = NC * NS
CHUNK = 128
NBUF = 8
TOT_CHUNK = 2560
E_PAD = TOT_CHUNK * CHUNK
K0S, K1S = 120, 40
K0D, K1D = 96, 64
ROWS = 632
ACC_N = ROWS * NS
TAIL = N - (NS - 1) * ROWS
DEGW = 8

_mesh = plsc.VectorSubcoreMesh(core_axis_name="c", subcore_axis_name="s")
_sc_params = pltpu.CompilerParams(use_tc_tiling_on_sc=False)


def _copy_out(acc, out_hbm, c, s):
  @pl.when(s < NS - 1)
  def _():
    pltpu.sync_copy(acc.at[pl.ds(s * ROWS, ROWS)],
                    out_hbm.at[c, pl.ds(s * ROWS, ROWS)])

  @pl.when(s == NS - 1)
  def _():
    pltpu.sync_copy(acc.at[pl.ds((NS - 1) * ROWS, TAIL)],
                    out_hbm.at[c, pl.ds((NS - 1) * ROWS, TAIL)])


def _make_deg_kernel():
  @functools.partial(
      pl.kernel,
      out_type=jax.ShapeDtypeStruct((NC, N, DEGW), jnp.float32),
      mesh=_mesh,
      compiler_params=_sc_params,
      name="sc_deg",
      scratch_types=[
          pltpu.VMEM((K0D, CHUNK), jnp.int32),
          pltpu.VMEM((CHUNK, DEGW), jnp.float32),
          pltpu.VMEM_SHARED((ACC_N, DEGW), jnp.float32),
          pltpu.SemaphoreType.DMA,
      ],
  )
  def deg_kernel(dst_hbm, ones_hbm, zeros_hbm, out_hbm, dst_v, ones_v, acc,
                 ssem):
    c = lax.axis_index("c")
    s = lax.axis_index("s")
    pltpu.sync_copy(zeros_hbm, acc.at[pl.ds(s * ROWS, ROWS)])
    pltpu.sync_copy(ones_hbm, ones_v)

    def run(base, nchunk):
      pltpu.sync_copy(dst_hbm.at[pl.ds(base, nchunk)],
                      dst_v.at[pl.ds(0, nchunk)])
      plsc.subcore_barrier()
      ng = nchunk // NBUF

      def body(g, carry):
        for b in range(NBUF):
          pltpu.async_copy(ones_v, acc.at[dst_v.at[g * NBUF + b]], ssem,
                           add=True)
        for b in range(NBUF):
          pltpu.make_async_copy(ones_v, acc.at[pl.ds(0, CHUNK)], ssem).wait()
        return carry

      lax.fori_loop(0, ng, body, 0)

    @pl.when(c == 0)
    def _():
      run(s * K0D, K0D)

    @pl.when(c == 1)
    def _():
      run(NS * K0D + s * K1D, K1D)

    plsc.subcore_barrier()
    _copy_out(acc, out_hbm, c, s)

  return deg_kernel


def _make_scatter_kernel(d: int):
  @functools.partial(
      pl.kernel,
      out_type=jax.ShapeDtypeStruct((NC, N, d), jnp.float32),
      mesh=_mesh,
      compiler_params=_sc_params,
      name=f"sc_scat{d}",
      scratch_types=[
          pltpu.VMEM((K0S, CHUNK), jnp.int32),
          pltpu.VMEM((K0S, CHUNK), jnp.int32),
          pltpu.VMEM((NBUF, CHUNK, d), jnp.float32),
          pltpu.VMEM_SHARED((ACC_N, d), jnp.float32),
      ] + [pltpu.SemaphoreType.DMA] * (2 * NBUF),
  )
  def scat_kernel(z_hbm, src_hbm, dst_hbm, zeros_hbm, out_hbm,
                  src_v, dst_v, rows_v, acc, *sems):
    gsems, ssems = sems[:NBUF], sems[NBUF:]
    c = lax.axis_index("c")
    s = lax.axis_index("s")
    pltpu.sync_copy(zeros_hbm, acc.at[pl.ds(s * ROWS, ROWS)])

    def gather_start(b, j):
      pltpu.async_copy(z_hbm.at[src_v.at[j]], rows_v.at[b], gsems[b])

    def gather_wait(b):
      pltpu.make_async_copy(z_hbm.at[pl.ds(0, CHUNK)], rows_v.at[b],
                            gsems[b]).wait()

    def scatter_start(b, j):
      pltpu.async_copy(rows_v.at[b], acc.at[dst_v.at[j]], ssems[b], add=True)

    def scatter_wait(b):
      pltpu.make_async_copy(rows_v.at[b], acc.at[pl.ds(0, CHUNK)],
                            ssems[b]).wait()

    def run(base, nchunk):
      pltpu.sync_copy(src_hbm.at[pl.ds(base, nchunk)],
                      src_v.at[pl.ds(0, nchunk)])
      pltpu.sync_copy(dst_hbm.at[pl.ds(base, nchunk)],
                      dst_v.at[pl.ds(0, nchunk)])
      plsc.subcore_barrier()
      ng = nchunk // NBUF
      for b in range(NBUF):
        gather_start(b, b)

      def body(g, carry):
        for b in range(NBUF):
          gather_wait(b)
          scatter_start(b, g * NBUF + b)
        for b in range(NBUF):
          scatter_wait(b)
          gather_start(b, (g + 1) * NBUF + b)
        return carry

      lax.fori_loop(0, ng - 1, body, 0)
      for b in range(NBUF):
        gather_wait(b)
        scatter_start(b, (ng - 1) * NBUF + b)
      for b in range(NBUF):
        scatter_wait(b)

    @pl.when(c == 0)
    def _():
      run(s * K0S, K0S)

    @pl.when(c == 1)
    def _():
      run(NS * K0S + s * K1S, K1S)

    plsc.subcore_barrier()
    _copy_out(acc, out_hbm, c, s)

  return scat_kernel


_deg_kernel = _make_deg_kernel()
_scat32 = _make_scatter_kernel(D_HID)
_scat16 = _make_scatter_kernel(D_OUT)

_HIGH = lax.Precision.HIGHEST


def _tc1(x_ref, w1_ref, deg_ref, z1_ref, dinv_ref):
  deg = deg_ref[0, :, :1] + deg_ref[1, :, :1] + 1.0
  dinv = lax.rsqrt(deg)
  y1 = jnp.dot(x_ref[...], w1_ref[...],
               preferred_element_type=jnp.float32, precision=_HIGH)
  z1_ref[...] = y1 * dinv
  dinv_ref[...] = dinv


def _tc2(s1_ref, z1_ref, dinv_ref, b1_ref, w2_ref, z2_ref):
  dinv = dinv_ref[...]
  h = dinv * (s1_ref[0] + s1_ref[1] + z1_ref[...]) + b1_ref[...]
  h = jnp.maximum(h, 0.0)
  y2 = jnp.dot(h, w2_ref[...],
               preferred_element_type=jnp.float32, precision=_HIGH)
  z2_ref[...] = y2 * dinv


def _tc3(s2_ref, z2_ref, dinv_ref, b2_ref, out_ref):
  out_ref[...] = (dinv_ref[...] * (s2_ref[0] + s2_ref[1] + z2_ref[...])
                  + b2_ref[...])


def kernel(x, edge_index, W1, b1, W2, b2):
  src = edge_index[0]
  dst = edge_index[1]
  pad = E_PAD - E
  src_p = jnp.concatenate([src, jnp.zeros((pad,), jnp.int32)])
  dst_p = jnp.concatenate([dst, jnp.full((pad,), N, jnp.int32)])
  src_b = src_p.reshape(TOT_CHUNK, CHUNK)
  dst_b = dst_p.reshape(TOT_CHUNK, CHUNK)

  ones_c = jnp.ones((CHUNK, DEGW), jnp.float32)
  zeros1 = jnp.zeros((ROWS, DEGW), jnp.float32)
  zeros32 = jnp.zeros((ROWS, D_HID), jnp.float32)
  zeros16 = jnp.zeros((ROWS, D_OUT), jnp.float32)

  deg_parts = _deg_kernel(dst_b, ones_c, zeros1)

  z1, dinv = pl.pallas_call(
      _tc1,
      out_shape=(jax.ShapeDtypeStruct((N, D_HID), jnp.float32),
                 jax.ShapeDtypeStruct((N, 1), jnp.float32)),
  )(x, W1, deg_parts)

  s1_parts = _scat32(z1, src_b, dst_b, zeros32)

  z2 = pl.pallas_call(
      _tc2,
      out_shape=jax.ShapeDtypeStruct((N, D_OUT), jnp.float32),
  )(s1_parts, z1, dinv, b1, W2)

  s2_parts = _scat16(z2, src_b, dst_b, zeros16)

  out = pl.pallas_call(
      _tc3,
      out_shape=jax.ShapeDtypeStruct((N, D_OUT), jnp.float32),
  )(s2_parts, z2, dinv, b2)

  return out
